# baseline (speedup 1.0000x reference)
import functools

import jax
import jax.numpy as jnp
from jax import lax
from jax.experimental import pallas as pl
from jax.experimental.pallas import tpu as pltpu
from jax.experimental.pallas import tpu_sc as plsc

_LANES = 16


def _emb_kernel_body(Bp, BC, F, D, num_cores,
                     ids_hbm, vals_hbm, table_hbm, out_hbm,
                     idx_v, val_v, rows_v, out_v, gsem, osem):
    wid = lax.axis_index("s") * num_cores + lax.axis_index("c")
    b_base = wid * Bp
    iota = lax.iota(jnp.int32, _LANES)

    def gather_desc(f, u):
        return pltpu.make_async_copy(
            table_hbm.at[idx_v.at[f]],
            rows_v.at[pl.ds(u * BC, BC)], gsem)

    def out_desc(f, u, b0):
        return pltpu.make_async_copy(
            out_v.at[u],
            out_hbm.at[f, :, pl.ds(b0, BC)], osem)

    def chunk_body(c, carry):
        b0 = b_base + c * BC
        pltpu.sync_copy(ids_hbm.at[:, pl.ds(b0, BC)], idx_v)
        pltpu.sync_copy(vals_hbm.at[:, pl.ds(b0, BC)], val_v)

        gather_desc(0, 0).start()

        def pair_body(g, c1):
            for u in range(2):
                f = 2 * g + u
                gather_desc(f, u).wait()

                @pl.when(f + 1 < F)
                def _():
                    gather_desc(f + 1, 1 - u).start()

                @pl.when(g > 0)
                def _():
                    out_desc(f - 2, u, b0).wait()

                def jb_body(jb, c2, u=u, f=f):
                    j0 = jb * _LANES
                    rowv = u * BC + j0 + iota
                    jv = j0 + iota
                    uv = jnp.full((_LANES,), u, jnp.int32)
                    vv = val_v[f, pl.ds(j0, _LANES)]
                    for k in range(D):
                        dv = (iota + k) % D
                        col = plsc.load_gather(rows_v, [rowv, dv])
                        plsc.store_scatter(out_v, [uv, dv, jv], col * vv)
                    return c2

                lax.fori_loop(0, BC // _LANES, jb_body, 0)
                out_desc(f, u, b0).start()
            return c1

        lax.fori_loop(0, F // 2, pair_body, 0)
        out_desc(F - 2, 0, b0).wait()
        out_desc(F - 1, 1, b0).wait()
        return carry

    lax.fori_loop(0, Bp // BC, chunk_body, 0)


def kernel(feature_id, feature_val, embedding_weight):
    B, F = feature_id.shape
    V, D = embedding_weight.shape
    ids_t = feature_id.T.astype(jnp.int32) * 4
    vals_t = feature_val.T
    table4 = jnp.broadcast_to(
        embedding_weight.reshape(V, 1, D), (V, 4, D)).reshape(4 * V, D)

    info = plsc.get_sparse_core_info()
    NW = info.num_cores * info.num_subcores
    Bp = B // NW
    BC = 128

    mesh = plsc.VectorSubcoreMesh(core_axis_name="c", subcore_axis_name="s")
    body = functools.partial(_emb_kernel_body, Bp, BC, F, D, info.num_cores)
    emb = pl.kernel(
        body,
        mesh=mesh,
        compiler_params=pltpu.CompilerParams(
            use_tc_tiling_on_sc=False, needs_layout_passes=False),
        out_type=jax.ShapeDtypeStruct((F, D, B), jnp.float32),
        scratch_types=[
            pltpu.VMEM((F, BC), jnp.int32),
            pltpu.VMEM((F, BC), jnp.float32),
            pltpu.VMEM((2 * BC, D), jnp.float32),
            pltpu.VMEM((2, D, BC), jnp.float32),
            pltpu.SemaphoreType.DMA,
            pltpu.SemaphoreType.DMA,
        ],
    )
    out_t = emb(ids_t, vals_t, table4)
    return jnp.transpose(out_t, (2, 0, 1))


# speedup vs baseline: 3.6175x; 3.6175x over previous
import functools

import jax
import jax.numpy as jnp
from jax import lax
from jax.experimental import pallas as pl
from jax.experimental.pallas import tpu as pltpu
from jax.experimental.pallas import tpu_sc as plsc

_LANES = 16


def _emb_kernel_body(Bp, BC, F, D, num_cores,
                     ids_hbm, vals_hbm, table_hbm, out_hbm,
                     idx_v, val_v, rows_v, out_v, gsem, osem):
    wid = lax.axis_index("s") * num_cores + lax.axis_index("c")
    b_base = wid * Bp
    iota = lax.iota(jnp.int32, _LANES)

    def gather_desc(f, u):
        return pltpu.make_async_copy(
            table_hbm.at[idx_v.at[f]],
            rows_v.at[pl.ds(u * BC, BC)], gsem)

    def out_desc(f, u, b0):
        return pltpu.make_async_copy(
            out_v.at[u],
            out_hbm.at[f, :, pl.ds(b0, BC)], osem)

    def chunk_body(c, carry):
        b0 = b_base + c * BC
        pltpu.sync_copy(ids_hbm.at[:, pl.ds(b0, BC)], idx_v)
        pltpu.sync_copy(vals_hbm.at[:, pl.ds(b0, BC)], val_v)

        gather_desc(0, 0).start()

        def pair_body(g, c1):
            for u in range(2):
                f = 2 * g + u
                gather_desc(f, u).wait()

                @pl.when(f + 1 < F)
                def _():
                    gather_desc(f + 1, 1 - u).start()

                @pl.when(g > 0)
                def _():
                    out_desc(f - 2, u, b0).wait()

                def jb_body(jb, c2, u=u, f=f):
                    j0 = jb * _LANES
                    rowv = u * BC + j0 + iota
                    jv = j0 + iota
                    uv = jnp.full((_LANES,), u, jnp.int32)
                    vv = val_v[f, pl.ds(j0, _LANES)]
                    for k in range(D):
                        dv = (iota + k) % D
                        col = plsc.load_gather(rows_v, [rowv, dv])
                        plsc.store_scatter(out_v, [uv, dv, jv], col * vv)
                    return c2

                lax.fori_loop(0, BC // _LANES, jb_body, 0)
                out_desc(f, u, b0).start()
            return c1

        lax.fori_loop(0, F // 2, pair_body, 0)
        out_desc(F - 2, 0, b0).wait()
        out_desc(F - 1, 1, b0).wait()
        return carry

    lax.fori_loop(0, Bp // BC, chunk_body, 0)


def kernel(feature_id, feature_val, embedding_weight):
    B, F = feature_id.shape
    V, D = embedding_weight.shape
    ids_t = feature_id.T.astype(jnp.int32)
    vals_t = feature_val.T

    info = plsc.get_sparse_core_info()
    NW = info.num_cores * info.num_subcores
    Bp = B // NW
    BC = 128

    mesh = plsc.VectorSubcoreMesh(core_axis_name="c", subcore_axis_name="s")
    body = functools.partial(_emb_kernel_body, Bp, BC, F, D, info.num_cores)
    emb = pl.kernel(
        body,
        mesh=mesh,
        compiler_params=pltpu.CompilerParams(
            use_tc_tiling_on_sc=False, needs_layout_passes=False),
        out_type=jax.ShapeDtypeStruct((F, D, B), jnp.float32),
        scratch_types=[
            pltpu.VMEM((F, BC), jnp.int32),
            pltpu.VMEM((F, BC), jnp.float32),
            pltpu.VMEM((2 * BC, D), jnp.float32),
            pltpu.VMEM((2, D, BC), jnp.float32),
            pltpu.SemaphoreType.DMA,
            pltpu.SemaphoreType.DMA,
        ],
    )
    out_t = emb(ids_t, vals_t, embedding_weight)
    return jnp.transpose(out_t, (2, 0, 1))
